# baseline (speedup 1.0000x reference)
import functools

import jax
import jax.numpy as jnp
from jax import lax
from jax.experimental import pallas as pl
from jax.experimental.pallas import tpu as pltpu
from jax.experimental.pallas import tpu_sc as plsc

ITEMS = 320000
SEG = 10000
D = 128
HALF = 64
NTILES = 16
LANES = 16
PER_TILE = ITEMS // NTILES
NBUF = 5
BLK = 250
NBLK = PER_TILE // BLK
CH = 125
NCH = BLK // CH
SEG_PER_TILE = SEG // NTILES
SEG_A = 624
DIVN = 640
DCH = 128


@functools.partial(
    pl.kernel,
    out_type=jax.ShapeDtypeStruct((SEG, D), jnp.float32),
    mesh=plsc.VectorSubcoreMesh(core_axis_name="c", subcore_axis_name="s"),
    scratch_types=[
        pltpu.VMEM_SHARED((SEG, HALF), jnp.float32),
        pltpu.VMEM_SHARED((SEG,), jnp.float32),
        pltpu.VMEM((NBUF, BLK, HALF), jnp.float32),
        pltpu.VMEM((NBUF, NCH, CH), jnp.int32),
        pltpu.VMEM((DCH,), jnp.float32),
        pltpu.VMEM((2, DCH), jnp.float32),
        pltpu.SemaphoreType.DMA((NBUF,)),
        pltpu.SemaphoreType.DMA((NBUF,)),
        pltpu.SemaphoreType.DMA,
    ],
    compiler_params=pltpu.CompilerParams(use_tc_tiling_on_sc=False),
)
def _seg_mean(y_hbm, emap_hbm, out_hbm, acc, cnt, rows, idx, ones, cntv,
              gsem, ssem, zsem):
    cid = lax.axis_index("c")
    sid = lax.axis_index("s")
    col0 = cid * HALF

    zero = jnp.zeros((LANES,), jnp.float32)
    one = jnp.ones((LANES,), jnp.float32)

    item0 = sid * PER_TILE
    erow0 = item0 // CH

    def start_gather(b, k):
        base = item0 + k * BLK
        for h in range(NCH):
            pltpu.async_copy(
                y_hbm.at[pl.ds(base + h * CH, CH), pl.ds(col0, HALF)],
                rows.at[b, pl.ds(h * CH, CH)], gsem.at[b])
        pltpu.async_copy(
            emap_hbm.at[pl.ds(erow0 + k * NCH, NCH)], idx.at[b], gsem.at[b])

    def wait_gather(b):
        for h in range(NCH):
            pltpu.make_async_copy(
                y_hbm.at[pl.ds(0, CH), pl.ds(col0, HALF)],
                rows.at[b, pl.ds(h * CH, CH)], gsem.at[b]).wait()
        pltpu.make_async_copy(
            emap_hbm.at[pl.ds(0, NCH)], idx.at[b], gsem.at[b]).wait()

    for b in range(1, NBUF):
        start_gather(b, b - 1)

    @pl.loop(0, CH)
    def _(r):
        for j in range(HALF // LANES):
            rows[0, r, pl.ds(j * LANES, LANES)] = zero

    for j in range(DCH // LANES):
        ones[pl.ds(j * LANES, LANES)] = zero

    seg0 = sid * SEG_A
    zdescs = []
    for off in range(0, SEG_PER_TILE, CH):
        n = min(CH, SEG_PER_TILE - off)
        zdescs.append((rows.at[0, pl.ds(0, n)],
                       acc.at[pl.ds(sid * SEG_PER_TILE + off, n)]))
    for off in range(0, DIVN, DCH):
        zdescs.append((ones.at[pl.ds(0, DCH)], cnt.at[pl.ds(seg0 + off, DCH)]))
    for src, dst in zdescs:
        pltpu.async_copy(src, dst, zsem)
    for src, dst in zdescs:
        pltpu.make_async_copy(src, dst, zsem).wait()

    for j in range(DCH // LANES):
        ones[pl.ds(j * LANES, LANES)] = one

    plsc.subcore_barrier()

    def fire_scatters(b):
        for j in range(NCH):
            pltpu.async_copy(
                rows.at[b, pl.ds(j * CH, CH)], acc.at[idx.at[b, j]],
                ssem.at[b], add=True)
            pltpu.async_copy(ones.at[pl.ds(0, CH)], cnt.at[idx.at[b, j]],
                             ssem.at[b], add=True)

    def drain_scatters(b):
        for j in range(NCH):
            pltpu.make_async_copy(
                rows.at[b, pl.ds(j * CH, CH)], acc.at[idx.at[b, j]],
                ssem.at[b]).wait()
            pltpu.make_async_copy(ones.at[pl.ds(0, CH)], cnt.at[idx.at[b, j]],
                                  ssem.at[b]).wait()

    @pl.loop(0, NBLK // NBUF)
    def _(kk):
        for b in range(NBUF):
            k = kk * NBUF + b
            bk = (b + 1) % NBUF
            nxt = b
            wait_gather(bk)

            @pl.when(k > 0)
            def _():
                drain_scatters(nxt)

            @pl.when(k + NBUF - 1 < NBLK)
            def _():
                start_gather(nxt, k + NBUF - 1)

            fire_scatters(bk)

    drain_scatters((NBLK - 1 + 1) % NBUF)
    plsc.subcore_barrier()

    def fire_div_load(c, p):
        base = seg0 + c * DCH
        pltpu.async_copy(acc.at[pl.ds(base, DCH)], rows.at[p, pl.ds(0, DCH)],
                         gsem.at[p])
        pltpu.async_copy(cnt.at[pl.ds(base, DCH)], cntv.at[p], gsem.at[p])

    def wait_div_load(c, p):
        base = seg0 + c * DCH
        pltpu.make_async_copy(acc.at[pl.ds(base, DCH)],
                              rows.at[p, pl.ds(0, DCH)], gsem.at[p]).wait()
        pltpu.make_async_copy(cnt.at[pl.ds(base, DCH)], cntv.at[p],
                              gsem.at[p]).wait()

    def div_write_descr(c, p):
        return pltpu.make_async_copy(
            rows.at[p, pl.ds(0, DCH)],
            out_hbm.at[pl.ds(seg0 + c * DCH, DCH), pl.ds(col0, HALF)],
            ssem.at[p])

    NDCH = DIVN // DCH
    fire_div_load(0, 0)
    for c in range(NDCH):
        p = c % 2
        wait_div_load(c, p)
        if c >= 1:
            div_write_descr(c - 1, (c - 1) % 2).wait()
        if c + 1 < NDCH:
            fire_div_load(c + 1, (c + 1) % 2)

        @pl.loop(0, DCH // LANES)
        def _(g):
            g16 = pl.multiple_of(g * LANES, LANES)
            c16 = cntv[p, pl.ds(g16, LANES)]
            inv16 = jnp.where(c16 > 0.0, 1.0 / jnp.maximum(c16, 1.0), 0.0)
            for i in range(LANES):
                f = inv16[i]
                for j in range(HALF // LANES):
                    rows[p, g16 + i, pl.ds(j * LANES, LANES)] = (
                        rows[p, g16 + i, pl.ds(j * LANES, LANES)] * f)

        div_write_descr(c, p).start()

    div_write_descr(NDCH - 1, (NDCH - 1) % 2).wait()


def kernel(X_in, Y, e_map, v_count):
    emap = e_map.astype(jnp.int32).reshape(ITEMS // CH, CH)
    return _seg_mean(Y, emap)


# speedup vs baseline: 1.0037x; 1.0037x over previous
import functools

import jax
import jax.numpy as jnp
from jax import lax
from jax.experimental import pallas as pl
from jax.experimental.pallas import tpu as pltpu
from jax.experimental.pallas import tpu_sc as plsc

ITEMS = 320000
SEG = 10000
D = 128
HALF = 64
NTILES = 16
LANES = 16
PER_TILE = ITEMS // NTILES
NBUF = 5
BLK = 250
NBLK = PER_TILE // BLK
CH = 125
NCH = BLK // CH
SEG_PER_TILE = SEG // NTILES
SEG_A = 624
DIVN = 640
DCH = 128


@functools.partial(
    pl.kernel,
    out_type=jax.ShapeDtypeStruct((SEG, D), jnp.float32),
    mesh=plsc.VectorSubcoreMesh(core_axis_name="c", subcore_axis_name="s"),
    scratch_types=[
        pltpu.VMEM_SHARED((SEG, HALF), jnp.float32),
        pltpu.VMEM_SHARED((SEG,), jnp.float32),
        pltpu.VMEM((NBUF, BLK, HALF), jnp.float32),
        pltpu.VMEM((NBUF, NCH, CH), jnp.int32),
        pltpu.VMEM((DCH,), jnp.float32),
        pltpu.VMEM((2, DCH), jnp.float32),
        pltpu.SemaphoreType.DMA((NBUF,)),
        pltpu.SemaphoreType.DMA((NBUF,)),
        pltpu.SemaphoreType.DMA,
    ],
    compiler_params=pltpu.CompilerParams(use_tc_tiling_on_sc=False),
)
def _seg_mean(y_hbm, emap_hbm, out_hbm, acc, cnt, rows, idx, ones, cntv,
              gsem, ssem, zsem):
    cid = lax.axis_index("c")
    sid = lax.axis_index("s")
    col0 = cid * HALF

    zero = jnp.zeros((LANES,), jnp.float32)
    one = jnp.ones((LANES,), jnp.float32)

    item0 = sid * PER_TILE
    erow0 = item0 // CH

    def start_gather(b, k):
        base = item0 + k * BLK
        pltpu.async_copy(
            y_hbm.at[pl.ds(base, BLK), pl.ds(col0, HALF)], rows.at[b], gsem.at[b])
        pltpu.async_copy(
            emap_hbm.at[pl.ds(erow0 + k * NCH, NCH)], idx.at[b], gsem.at[b])

    def wait_gather(b):
        pltpu.make_async_copy(
            y_hbm.at[pl.ds(0, BLK), pl.ds(col0, HALF)], rows.at[b], gsem.at[b]).wait()
        pltpu.make_async_copy(
            emap_hbm.at[pl.ds(0, NCH)], idx.at[b], gsem.at[b]).wait()

    for b in range(1, NBUF):
        start_gather(b, b - 1)

    @pl.loop(0, CH)
    def _(r):
        for j in range(HALF // LANES):
            rows[0, r, pl.ds(j * LANES, LANES)] = zero

    for j in range(DCH // LANES):
        ones[pl.ds(j * LANES, LANES)] = zero

    seg0 = sid * SEG_A
    zdescs = []
    for off in range(0, SEG_PER_TILE, CH):
        n = min(CH, SEG_PER_TILE - off)
        zdescs.append((rows.at[0, pl.ds(0, n)],
                       acc.at[pl.ds(sid * SEG_PER_TILE + off, n)]))
    for off in range(0, DIVN, DCH):
        zdescs.append((ones.at[pl.ds(0, DCH)], cnt.at[pl.ds(seg0 + off, DCH)]))
    for src, dst in zdescs:
        pltpu.async_copy(src, dst, zsem)
    for src, dst in zdescs:
        pltpu.make_async_copy(src, dst, zsem).wait()

    for j in range(DCH // LANES):
        ones[pl.ds(j * LANES, LANES)] = one

    plsc.subcore_barrier()

    def fire_scatters(b):
        for j in range(NCH):
            pltpu.async_copy(
                rows.at[b, pl.ds(j * CH, CH)], acc.at[idx.at[b, j]],
                ssem.at[b], add=True)
            pltpu.async_copy(ones.at[pl.ds(0, CH)], cnt.at[idx.at[b, j]],
                             ssem.at[b], add=True)

    def drain_scatters(b):
        for j in range(NCH):
            pltpu.make_async_copy(
                rows.at[b, pl.ds(j * CH, CH)], acc.at[idx.at[b, j]],
                ssem.at[b]).wait()
            pltpu.make_async_copy(ones.at[pl.ds(0, CH)], cnt.at[idx.at[b, j]],
                                  ssem.at[b]).wait()

    @pl.loop(0, NBLK // NBUF)
    def _(kk):
        for b in range(NBUF):
            k = kk * NBUF + b
            bk = (b + 1) % NBUF
            nxt = b
            wait_gather(bk)

            @pl.when(k > 0)
            def _():
                drain_scatters(nxt)

            @pl.when(k + NBUF - 1 < NBLK)
            def _():
                start_gather(nxt, k + NBUF - 1)

            fire_scatters(bk)

    drain_scatters((NBLK - 1 + 1) % NBUF)
    plsc.subcore_barrier()

    def fire_div_load(c, p):
        base = seg0 + c * DCH
        pltpu.async_copy(acc.at[pl.ds(base, DCH)], rows.at[p, pl.ds(0, DCH)],
                         gsem.at[p])
        pltpu.async_copy(cnt.at[pl.ds(base, DCH)], cntv.at[p], gsem.at[p])

    def wait_div_load(c, p):
        base = seg0 + c * DCH
        pltpu.make_async_copy(acc.at[pl.ds(base, DCH)],
                              rows.at[p, pl.ds(0, DCH)], gsem.at[p]).wait()
        pltpu.make_async_copy(cnt.at[pl.ds(base, DCH)], cntv.at[p],
                              gsem.at[p]).wait()

    def div_write_descr(c, p):
        return pltpu.make_async_copy(
            rows.at[p, pl.ds(0, DCH)],
            out_hbm.at[pl.ds(seg0 + c * DCH, DCH), pl.ds(col0, HALF)],
            ssem.at[p])

    NDCH = DIVN // DCH
    fire_div_load(0, 0)
    for c in range(NDCH):
        p = c % 2
        wait_div_load(c, p)
        if c >= 1:
            div_write_descr(c - 1, (c - 1) % 2).wait()
        if c + 1 < NDCH:
            fire_div_load(c + 1, (c + 1) % 2)

        @pl.loop(0, DCH // LANES)
        def _(g):
            g16 = pl.multiple_of(g * LANES, LANES)
            c16 = cntv[p, pl.ds(g16, LANES)]
            inv16 = jnp.where(c16 > 0.0, 1.0 / jnp.maximum(c16, 1.0), 0.0)
            for i in range(LANES):
                f = inv16[i]
                for j in range(HALF // LANES):
                    rows[p, g16 + i, pl.ds(j * LANES, LANES)] = (
                        rows[p, g16 + i, pl.ds(j * LANES, LANES)] * f)

        div_write_descr(c, p).start()

    div_write_descr(NDCH - 1, (NDCH - 1) % 2).wait()


def kernel(X_in, Y, e_map, v_count):
    emap = e_map.astype(jnp.int32).reshape(ITEMS // CH, CH)
    return _seg_mean(Y, emap)
